# Initial kernel scaffold; baseline (speedup 1.0000x reference)
import jax
import jax.numpy as jnp
from jax.experimental import pallas as pl


def kernel(x, edge_index, batch, W0, b0, g0, be0, W1, b1, g1, be1, W2, b2):
    raise NotImplementedError("write your pallas kernel here")


# speedup vs baseline: 57.1140x; 57.1140x over previous
import functools
import jax
import jax.numpy as jnp
from jax.experimental import pallas as pl
from jax.experimental.pallas import tpu as pltpu
from jax.experimental.pallas import tpu_sc as plsc

N = 10000
NPAD = 10240
D = 128
H = 16
C = 10
E = 320000
G = 64
EPS = 1e-5

NC = 2
NS = 16
NW = NC * NS
BLK = 128
EPT = E // NW
SBLK = 2000
NSB = EPT // SBLK
ROWS_PT = NPAD // NS


def _sc_degree_body(dstI, out, dst_v, ones_v, zb_v, deg_sh):
    c = jax.lax.axis_index("core")
    s = jax.lax.axis_index("subcore")
    w = c * NS + s

    @pl.loop(0, SBLK // 16)
    def _(i):
        ones_v[pl.ds(i * 16, 16)] = jnp.ones((16,), jnp.float32)

    @pl.loop(0, ROWS_PT // 16)
    def _(i):
        zb_v[pl.ds(i * 16, 16)] = jnp.zeros((16,), jnp.float32)

    pltpu.sync_copy(zb_v, deg_sh.at[pl.ds(s * ROWS_PT, ROWS_PT)])

    @pl.loop(0, NSB)
    def _(j):
        pltpu.sync_copy(dstI.at[pl.ds(pl.multiple_of(w * EPT + j * SBLK, 8), SBLK)],
                        dst_v.at[j])

    plsc.subcore_barrier()

    @pl.loop(0, NSB)
    def _(b):
        pltpu.sync_copy(ones_v, deg_sh.at[dst_v.at[b]], add=True)

    plsc.subcore_barrier()
    pltpu.sync_copy(deg_sh.at[pl.ds(s * ROWS_PT, ROWS_PT)],
                    out.at[c].at[pl.ds(s * ROWS_PT, ROWS_PT)])


def _sc_aggregate_body(ht, srcI, dstI, out,
                       src_v, dst_v, rows0, rows1,
                       gsem0, gsem1, agg_sh):
    c = jax.lax.axis_index("core")
    s = jax.lax.axis_index("subcore")
    w = c * NS + s
    r0 = s * ROWS_PT

    @pl.loop(0, BLK)
    def _(i):
        rows0[pl.ds(i, 1), :] = jnp.zeros((1, H), jnp.float32)

    zrows = rows0.at[pl.ds(0, BLK)]
    @pl.loop(0, ROWS_PT // BLK)
    def _(j):
        pltpu.sync_copy(zrows, agg_sh.at[pl.ds(r0 + j * BLK, BLK)])

    @pl.loop(0, NSB)
    def _(j):
        off = pl.multiple_of(w * EPT + j * SBLK, 8)
        pltpu.sync_copy(srcI.at[pl.ds(off, SBLK)], src_v.at[j])
        pltpu.sync_copy(dstI.at[pl.ds(off, SBLK)], dst_v.at[j])

    plsc.subcore_barrier()

    bufs = (rows0, rows1)
    sems = (gsem0, gsem1)
    pltpu.async_copy(ht.at[src_v.at[0]], rows0, gsem0)
    pltpu.async_copy(ht.at[src_v.at[1]], rows1, gsem1)
    for b in range(NSB):
        buf, sem = bufs[b % 2], sems[b % 2]
        pltpu.make_async_copy(ht.at[src_v.at[b]], buf, sem).wait()
        pltpu.sync_copy(buf, agg_sh.at[dst_v.at[b]], add=True)
        if b + 2 < NSB:
            pltpu.async_copy(ht.at[src_v.at[b + 2]], buf, sem)

    plsc.subcore_barrier()
    pltpu.sync_copy(agg_sh.at[pl.ds(s * ROWS_PT, ROWS_PT)],
                    out.at[c].at[pl.ds(s * ROWS_PT, ROWS_PT)])


@functools.lru_cache(maxsize=None)
def _sc_kernels():
    mesh = plsc.VectorSubcoreMesh(core_axis_name="core", subcore_axis_name="subcore")
    cp = pltpu.CompilerParams(use_tc_tiling_on_sc=False,
                              disable_bounds_checks=True)
    sc_degree = pl.kernel(
        _sc_degree_body,
        out_type=jax.ShapeDtypeStruct((NC, NPAD), jnp.float32),
        mesh=mesh,
        scratch_types=[
            pltpu.VMEM((NSB, SBLK), jnp.int32),
            pltpu.VMEM((SBLK,), jnp.float32),
            pltpu.VMEM((ROWS_PT,), jnp.float32),
            pltpu.VMEM_SHARED((NPAD,), jnp.float32),
        ],
        compiler_params=cp,
    )
    sc_aggregate = pl.kernel(
        _sc_aggregate_body,
        out_type=jax.ShapeDtypeStruct((NC, NPAD, H), jnp.float32),
        mesh=mesh,
        scratch_types=[
            pltpu.VMEM((NSB, SBLK), jnp.int32),
            pltpu.VMEM((NSB, SBLK), jnp.int32),
            pltpu.VMEM((SBLK, H), jnp.float32),
            pltpu.VMEM((SBLK, H), jnp.float32),
            pltpu.SemaphoreType.DMA,
            pltpu.SemaphoreType.DMA,
            pltpu.VMEM_SHARED((NPAD, H), jnp.float32),
        ],
        compiler_params=cp,
    )
    return sc_degree, sc_aggregate


def _tc1_body(degp_ref, x_ref, w0_ref, ht_ref, dinv_ref):
    dp = degp_ref[...]
    deg = dp[0:1, :] + dp[1:2, :] + 1.0
    dinv = jnp.transpose(jax.lax.rsqrt(deg), (1, 0))
    dinv_ref[...] = dinv
    h = jnp.dot(x_ref[...], w0_ref[...], preferred_element_type=jnp.float32)
    ht_ref[0:N, :] = h * dinv[0:N]
    ht_ref[N:NPAD, :] = jnp.zeros((NPAD - N, H), jnp.float32)


def _tc_mid_body(aggp_ref, htp_ref, dinv_ref, b_ref, g_ref, be_ref, w_ref, out_ref):
    ap = aggp_ref[...]
    agg = ap[0] + ap[1]
    dinv = dinv_ref[...]
    z = (agg[0:N] + htp_ref[0:N, :]) * dinv[0:N] + b_ref[...]
    mean = jnp.mean(z, axis=0, keepdims=True)
    zc = z - mean
    var = jnp.mean(zc * zc, axis=0, keepdims=True)
    zn = zc * jax.lax.rsqrt(var + EPS) * g_ref[...] + be_ref[...]
    zr = jnp.maximum(zn, 0.0)
    h = jnp.dot(zr, w_ref[...], preferred_element_type=jnp.float32)
    out_ref[0:N, :] = h * dinv[0:N]
    out_ref[N:NPAD, :] = jnp.zeros((NPAD - N, H), jnp.float32)


def _tc_out_body(aggp_ref, htp_ref, dinv_ref, b_ref, batch_ref, out_ref):
    ap = aggp_ref[...]
    agg = ap[0] + ap[1]
    node = (agg[0:N] + htp_ref[0:N, :]) * dinv_ref[...][0:N] + b_ref[...]
    bat = batch_ref[...]
    oh = (jax.lax.broadcasted_iota(jnp.int32, (G, N), 0) == bat)
    ohf = oh.astype(jnp.float32)
    sums = jnp.dot(ohf, node, preferred_element_type=jnp.float32)
    cnt = jnp.sum(ohf, axis=1, keepdims=True)
    out_ref[...] = (sums / jnp.maximum(cnt, 1.0))[:, 0:C]


_f32 = jnp.float32

_tc1 = pl.pallas_call(
    _tc1_body,
    out_shape=(jax.ShapeDtypeStruct((NPAD, H), _f32),
               jax.ShapeDtypeStruct((NPAD, 1), _f32)),
)

_tc_mid = pl.pallas_call(
    _tc_mid_body,
    out_shape=jax.ShapeDtypeStruct((NPAD, H), _f32),
)

_tc_out = pl.pallas_call(
    _tc_out_body,
    out_shape=jax.ShapeDtypeStruct((G, C), _f32),
)


def kernel(x, edge_index, batch, W0, b0, g0, be0, W1, b1, g1, be1, W2, b2):
    srcP = edge_index[0].astype(jnp.int32)
    dstP = edge_index[1].astype(jnp.int32)

    sc_degree, sc_aggregate = _sc_kernels()
    degp = sc_degree(dstP)
    ht0, dinv = _tc1(degp, x, W0)
    agg0 = sc_aggregate(ht0, srcP, dstP)
    ht1 = _tc_mid(agg0, ht0, dinv, b0.reshape(1, H), g0.reshape(1, H),
                  be0.reshape(1, H), W1)
    agg1 = sc_aggregate(ht1, srcP, dstP)
    W2p = jnp.pad(W2, ((0, 0), (0, H - C)))
    ht2 = _tc_mid(agg1, ht1, dinv, b1.reshape(1, H), g1.reshape(1, H),
                  be1.reshape(1, H), W2p)
    agg2 = sc_aggregate(ht2, srcP, dstP)
    b2p = jnp.pad(b2, (0, H - C)).reshape(1, H)
    batch2d = batch.astype(jnp.int32).reshape(1, N)
    return _tc_out(agg2, ht2, dinv, b2p, batch2d)
